# baseline (speedup 1.0000x reference)
import jax
import jax.numpy as jnp
from jax import lax
from jax.experimental import pallas as pl
from jax.experimental.pallas import tpu as pltpu

_G = 66
_GR = 132
_PLR = 8960
_PB_L = 9216
_OUT_R = 8448


def _fused_kernel(r_ref, w1_ref, b1_ref, w2_ref, b2_ref, o_ref, pb_ref):
    s = r_ref[0]

    for u in range(_G):
        for p in range(2):
            pb_ref[p:p + 1, u * _GR:u * _GR + _GR] = (
                s[u:u + 1, p * 256:p * 256 + _GR])

    pt = jnp.concatenate(
        [pb_ref[0:2, a * _GR + kw:a * _GR + kw + _PLR]
         for a in range(2) for kw in range(4)], axis=0)

    a1 = jnp.dot(w1_ref[...], pt, preferred_element_type=jnp.float32)
    a1 = a1 + b1_ref[:, 0:1]
    a1 = jnp.where(a1 > 0, a1, 0.2 * a1)

    k = lax.broadcasted_iota(jnp.int32, (1, _PLR), 1)
    pos = k >> 1
    ii = pos // _G
    jj = pos - ii * _G
    mask = ((k & 1) == 0) & (ii >= 1) & (ii <= 64) & (jj >= 1) & (jj <= 64)
    a1 = jnp.where(mask, a1, 0.0)

    ut = jnp.dot(w2_ref[...], a1, preferred_element_type=jnp.float32)

    acc = b2_ref[0:1, 0:1] + jnp.zeros((1, _OUT_R), jnp.float32)
    for kh in range(4):
        for kw in range(4):
            t = kh * 4 + kw
            off = 2 * (kh * _G + kw)
            acc = acc + ut[t:t + 1, off:off + _OUT_R]
    o_ref[0] = 1.0 / (1.0 + jnp.exp(-acc))


def kernel(x, w1, b1, w2, b2):
    n = x.shape[0]

    xp = jnp.pad(x[:, 0], ((0, 0), (3, 13), (3, 125)))
    rows = xp.reshape(n, 72, 512)

    w1m = w1.reshape(64, 16)
    perm = [(2 * a + p) * 4 + kw
            for a in range(2) for kw in range(4) for p in range(2)]
    w1g = w1m[:, jnp.array(perm)]
    b1c = jnp.broadcast_to(b1.reshape(64, 1), (64, 128))
    w2c = jnp.transpose(w2.reshape(64, 16))
    b2c = jnp.broadcast_to(b2.reshape(1, 1), (8, 128))

    cost = pl.CostEstimate(
        flops=2 * n * _PLR * (64 * 16 + 16 * 64) + n * _OUT_R * 20,
        transcendentals=n * _OUT_R,
        bytes_accessed=4 * (n * 72 * 512 + n * _OUT_R + 2 * 64 * 16),
    )
    out = pl.pallas_call(
        _fused_kernel,
        out_shape=jax.ShapeDtypeStruct((n, 1, _OUT_R), jnp.float32),
        grid=(n,),
        in_specs=[
            pl.BlockSpec((1, 72, 512), lambda i: (i, 0, 0)),
            pl.BlockSpec((64, 16), lambda i: (0, 0)),
            pl.BlockSpec((64, 128), lambda i: (0, 0)),
            pl.BlockSpec((16, 64), lambda i: (0, 0)),
            pl.BlockSpec((8, 128), lambda i: (0, 0)),
        ],
        out_specs=pl.BlockSpec((1, 1, _OUT_R), lambda i: (i, 0, 0)),
        scratch_shapes=[pltpu.VMEM((8, _PB_L), jnp.float32)],
        compiler_params=pltpu.CompilerParams(
            dimension_semantics=("parallel",)),
        cost_estimate=cost,
    )(rows, w1g, b1c, w2c, b2c)

    o = out[:, 0, 0:2 * 63 * _G:2].reshape(n, 63, _G)[:, :, :63]
    return o[:, None]


# speedup vs baseline: 2.1377x; 2.1377x over previous
import jax
import jax.numpy as jnp
from jax import lax
from jax.experimental import pallas as pl
from jax.experimental.pallas import tpu as pltpu

_G = 66
_P = _G * _G
_PL = 4480
_PB_L = 4608
_OUT_L = 4224


def _fused_kernel(pl_ref, w1_ref, b1_ref, w2_ref, b2_ref, o_ref, pb_ref):
    planes = pl_ref[0]

    for u in range(_G):
        pb_ref[0:4, u * _G:u * _G + _G] = planes[:, u, 0:_G]

    pb = pb_ref[0:4, :]
    pt = jnp.concatenate(
        [pb[:, d:d + _PL] for d in (0, 1, _G, _G + 1)], axis=0)

    a1 = jnp.dot(w1_ref[...], pt, preferred_element_type=jnp.float32)
    a1 = a1 + b1_ref[:, 0:1]
    a1 = jnp.where(a1 > 0, a1, 0.2 * a1)

    p = lax.broadcasted_iota(jnp.int32, (1, _PL), 1)
    ii = p // _G
    jj = p - ii * _G
    mask = (ii >= 1) & (ii <= 64) & (jj >= 1) & (jj <= 64)
    a1 = jnp.where(mask, a1, 0.0)

    ut = jnp.dot(w2_ref[...], a1, preferred_element_type=jnp.float32)

    acc = b2_ref[0:1, 0:1] + jnp.zeros((1, _OUT_L), jnp.float32)
    for kh in range(4):
        for kw in range(4):
            t = kh * 4 + kw
            off = kh * _G + kw
            acc = acc + ut[t:t + 1, off:off + _OUT_L]
    o_ref[0] = 1.0 / (1.0 + jnp.exp(-acc))


def kernel(x, w1, b1, w2, b2):
    n = x.shape[0]

    xp = jnp.pad(x[:, 0], ((0, 0), (3, 13), (3, 125)))
    planes = xp.reshape(n, 72, 2, 128, 2)
    planes = planes.transpose(0, 2, 4, 1, 3).reshape(n, 4, 72, 128)

    w1m = w1.reshape(64, 16)
    perm = [(2 * a + p) * 4 + (2 * b + r)
            for a in range(2) for b in range(2)
            for p in range(2) for r in range(2)]
    w1g = w1m[:, jnp.array(perm)]
    b1c = jnp.broadcast_to(b1.reshape(64, 1), (64, 128))
    w2c = jnp.transpose(w2.reshape(64, 16))
    b2c = jnp.broadcast_to(b2.reshape(1, 1), (8, 128))

    cost = pl.CostEstimate(
        flops=2 * n * _PL * (64 * 16 + 16 * 64) + n * _OUT_L * 20,
        transcendentals=n * _OUT_L,
        bytes_accessed=4 * (n * 72 * 512 + n * _OUT_L + 2 * 64 * 16),
    )
    out = pl.pallas_call(
        _fused_kernel,
        out_shape=jax.ShapeDtypeStruct((n, 1, _OUT_L), jnp.float32),
        grid=(n,),
        in_specs=[
            pl.BlockSpec((1, 4, 72, 128), lambda i: (i, 0, 0, 0)),
            pl.BlockSpec((64, 16), lambda i: (0, 0)),
            pl.BlockSpec((64, 128), lambda i: (0, 0)),
            pl.BlockSpec((16, 64), lambda i: (0, 0)),
            pl.BlockSpec((8, 128), lambda i: (0, 0)),
        ],
        out_specs=pl.BlockSpec((1, 1, _OUT_L), lambda i: (i, 0, 0)),
        scratch_shapes=[pltpu.VMEM((8, _PB_L), jnp.float32)],
        compiler_params=pltpu.CompilerParams(
            dimension_semantics=("parallel",)),
        cost_estimate=cost,
    )(planes, w1g, b1c, w2c, b2c)

    o = out[:, 0, :63 * _G].reshape(n, 63, _G)[:, :, :63]
    return o[:, None]
